# baseline (speedup 1.0000x reference)
import jax
import jax.numpy as jnp
from jax.experimental import pallas as pl
from jax.experimental.pallas import tpu as pltpu

_B, _C, _H, _W = 8, 96, 224, 224
_E = 64


def _body(idx_ref, emb_row_ref, img_ref, out_ref, scratch, sem_img, sem_emb):
    b = pl.program_id(0)
    pltpu.make_async_copy(
        img_ref.at[b], out_ref.at[b, _E:], sem_img.at[b]).start()

    slot = jax.lax.rem(b, 2)

    @pl.when(b >= 2)
    def _wait_prev():
        pltpu.make_async_copy(
            scratch.at[slot], out_ref.at[b - 2, : _E], sem_emb.at[slot]).wait()

    row = emb_row_ref[0, 0, :]
    scratch[slot] = jnp.broadcast_to(row[:, None, None], (_E, _H, _W))
    pltpu.make_async_copy(
        scratch.at[slot], out_ref.at[b, : _E], sem_emb.at[slot]).start()

    @pl.when(b == _B - 1)
    def _drain():
        pltpu.make_async_copy(
            scratch.at[0], out_ref.at[_B - 2, : _E], sem_emb.at[0]).wait()
        pltpu.make_async_copy(
            scratch.at[1], out_ref.at[_B - 1, : _E], sem_emb.at[1]).wait()
        for bb in range(_B):
            pltpu.make_async_copy(
                img_ref.at[bb], out_ref.at[bb, _E:], sem_img.at[bb]).wait()


def kernel(class_idx, image, emb_table):
    return pl.pallas_call(
        _body,
        grid_spec=pltpu.PrefetchScalarGridSpec(
            num_scalar_prefetch=1,
            grid=(_B,),
            in_specs=[
                pl.BlockSpec((1, 1, _E), lambda b, idx_ref: (idx_ref[b], 0, 0)),
                pl.BlockSpec(memory_space=pltpu.MemorySpace.HBM),
            ],
            out_specs=pl.BlockSpec(memory_space=pltpu.MemorySpace.HBM),
            scratch_shapes=[
                pltpu.VMEM((2, _E, _H, _W), jnp.float32),
                pltpu.SemaphoreType.DMA((_B,)),
                pltpu.SemaphoreType.DMA((2,)),
            ],
        ),
        out_shape=jax.ShapeDtypeStruct((_B, _C + _E, _H, _W), jnp.float32),
    )(class_idx, emb_table.reshape(-1, 1, _E), image)


# speedup vs baseline: 36.6150x; 36.6150x over previous
import jax
import jax.numpy as jnp
from jax.experimental import pallas as pl
from jax.experimental.pallas import tpu as pltpu

_B, _C, _H, _W = 8, 96, 224, 224
_E = 64
_HB = 112


def _body(idx_ref, emb_row_ref, img_ref, out_ref):
    row = emb_row_ref[0, 0, :]
    out_ref[0, :_E] = jnp.broadcast_to(row[:, None, None], (_E, _HB, _W))
    out_ref[0, _E:] = img_ref[0]


def kernel(class_idx, image, emb_table):
    grid = (_B, _H // _HB)
    return pl.pallas_call(
        _body,
        grid_spec=pltpu.PrefetchScalarGridSpec(
            num_scalar_prefetch=1,
            grid=grid,
            in_specs=[
                pl.BlockSpec((1, 1, _E), lambda b, h, idx_ref: (idx_ref[b], 0, 0)),
                pl.BlockSpec((1, _C, _HB, _W), lambda b, h, idx_ref: (b, 0, h, 0)),
            ],
            out_specs=pl.BlockSpec((1, _C + _E, _HB, _W),
                                   lambda b, h, idx_ref: (b, 0, h, 0)),
        ),
        out_shape=jax.ShapeDtypeStruct((_B, _C + _E, _H, _W), jnp.float32),
    )(class_idx, emb_table.reshape(-1, 1, _E), image)
